# Initial kernel scaffold; baseline (speedup 1.0000x reference)
import jax
import jax.numpy as jnp
from jax.experimental import pallas as pl


def kernel(x, edge_index, W1, b1, W2, b2):
    raise NotImplementedError("write your pallas kernel here")


# speedup vs baseline: 15.7671x; 15.7671x over previous
import functools

import jax
import jax.numpy as jnp
from jax import lax
from jax.experimental import pallas as pl
from jax.experimental.pallas import tpu as pltpu
from jax.experimental.pallas import tpu_sc as plsc

N = 10000
E = 320000
NC = 2
NS = 16
NW = NC * NS
EP = E // NW
CH = 80
NCH = EP // CH
NPAD = 10240
ROWS_PER_TILE = NPAD // NS

_mesh = plsc.VectorSubcoreMesh(core_axis_name="c", subcore_axis_name="s")
_sc_params = pltpu.CompilerParams(use_tc_tiling_on_sc=False)


def _zero_ref(ref, nwords):
    flat = nwords // 16

    def body(i, _):
        ref[pl.ds(i * 16, 16)] = jnp.zeros((16,), jnp.float32)
        return 0

    lax.fori_loop(0, flat, body, 0)


@functools.partial(
    pl.kernel,
    out_type=jax.ShapeDtypeStruct((NC * N,), jnp.float32),
    mesh=_mesh,
    compiler_params=_sc_params,
    scratch_types=[
        pltpu.VMEM((NCH, CH), jnp.int32),
        pltpu.VMEM((CH,), jnp.float32),
        pltpu.VMEM((1024,), jnp.float32),
        pltpu.VMEM_SHARED((N,), jnp.float32),
    ],
)
def _deg_kernel(dst_hbm, out_hbm, didx, ones_v, zbuf, deg_sp):
    c = lax.axis_index("c")
    s = lax.axis_index("s")
    wid = c * NS + s

    _zero_ref(zbuf, 1024)

    def fill_ones(i, _):
        ones_v[pl.ds(i * 16, 16)] = jnp.ones((16,), jnp.float32)
        return 0

    lax.fori_loop(0, CH // 16, fill_ones, 0)

    @pl.when(s < 10)
    def _():
        pltpu.sync_copy(zbuf.at[pl.ds(0, 1000)], deg_sp.at[pl.ds(s * 1000, 1000)])

    plsc.subcore_barrier()

    pltpu.sync_copy(dst_hbm.at[wid], didx)

    def body(j, _):
        pltpu.sync_copy(ones_v, deg_sp.at[didx.at[j]], add=True)
        return 0

    lax.fori_loop(0, NCH, body, 0)

    plsc.subcore_barrier()

    @pl.when(s < 10)
    def _():
        pltpu.sync_copy(deg_sp.at[pl.ds(s * 1000, 1000)],
                        zbuf.at[pl.ds(0, 1000)])
        pltpu.sync_copy(zbuf.at[pl.ds(0, 1000)],
                        out_hbm.at[pl.ds(c * N + s * 1000, 1000)])


AD = 64


def _make_agg(P):
    zr = 128

    @functools.partial(
        pl.kernel,
        out_type=jax.ShapeDtypeStruct((NC, P, NPAD, AD), jnp.float32),
        mesh=_mesh,
        compiler_params=_sc_params,
        scratch_types=[
            pltpu.VMEM((NCH, CH), jnp.int32),
            pltpu.VMEM((NCH, CH), jnp.int32),
            pltpu.VMEM((CH, AD), jnp.float32),
            pltpu.VMEM((zr, AD), jnp.float32),
            pltpu.VMEM_SHARED((NPAD, AD), jnp.float32),
            pltpu.SemaphoreType.DMA,
        ],
    )
    def agg(*args):
        ys = args[:P]
        src_hbm, dst_hbm, out_hbm, sidx, didx, rows, zbuf, acc, gsem = args[P:]
        c = lax.axis_index("c")
        s = lax.axis_index("s")
        wid = c * NS + s

        cols = AD // 16

        def zstore(t, _):
            zbuf[t // cols, pl.ds((t % cols) * 16, 16)] = jnp.zeros(
                (16,), jnp.float32)
            return 0

        lax.fori_loop(0, zr * cols, zstore, 0)

        pltpu.sync_copy(src_hbm.at[wid], sidx)
        pltpu.sync_copy(dst_hbm.at[wid], didx)

        for p in range(P):
            for k in range(ROWS_PER_TILE // zr):
                pltpu.sync_copy(
                    zbuf, acc.at[pl.ds(s * ROWS_PER_TILE + k * zr, zr)])
            plsc.subcore_barrier()

            y_hbm = ys[p]

            def body(j, _):
                pltpu.async_copy(y_hbm.at[sidx.at[j]], rows, gsem).wait()
                pltpu.sync_copy(rows, acc.at[didx.at[j]], add=True)
                return 0

            lax.fori_loop(0, NCH, body, 0)

            plsc.subcore_barrier()

            for k in range(ROWS_PER_TILE // zr):
                base = s * ROWS_PER_TILE + k * zr
                pltpu.sync_copy(acc.at[pl.ds(base, zr)], zbuf)
                pltpu.sync_copy(zbuf, out_hbm.at[c, p, pl.ds(base, zr)])
            if p + 1 < P:
                lax.fori_loop(0, zr * cols, zstore, 0)

    return agg


_agg2 = _make_agg(2)
_agg1 = _make_agg(1)


def _tc1_body(x_ref, w_ref, d0_ref, d1_ref, y_ref, dinv_ref):
    deg = d0_ref[...] + d1_ref[...] + 1.0
    dinv = lax.rsqrt(deg)
    dinv_ref[...] = dinv
    xw = jnp.dot(x_ref[...], w_ref[...],
                 preferred_element_type=jnp.float32,
                 precision=lax.Precision.HIGHEST)
    y_ref[...] = xw * dinv


_SELU_SCALE = 1.0507009873554804934193349852946
_SELU_ALPHA = 1.6732632423543772848170429916717


def _tc2_body(p0_ref, p1_ref, y1_ref, dinv_ref, b1_ref, w2_ref, y2_ref):
    dinv = dinv_ref[...]
    pre = dinv * (p0_ref[...] + p1_ref[...] + y1_ref[...]) + b1_ref[...]
    h = _SELU_SCALE * jnp.where(
        pre > 0.0, pre, _SELU_ALPHA * (jnp.exp(pre) - 1.0))
    hw = jnp.dot(h, w2_ref[...],
                 preferred_element_type=jnp.float32,
                 precision=lax.Precision.HIGHEST)
    y2_ref[...] = hw * dinv


def _tc3_body(q0_ref, q1_ref, y2_ref, dinv_ref, b2_ref, out_ref):
    o = dinv_ref[...] * (q0_ref[...] + q1_ref[...] + y2_ref[...]) + b2_ref[...]
    m = jnp.max(o, axis=1, keepdims=True)
    lse = m + jnp.log(jnp.sum(jnp.exp(o - m), axis=1, keepdims=True))
    out_ref[...] = o - lse


def kernel(x, edge_index, W1, b1, W2, b2):
    D_in = x.shape[1]
    D_h = W1.shape[1]
    D_out = W2.shape[1]

    src3 = edge_index[0].reshape(NW, NCH, CH)
    dst3 = edge_index[1].reshape(NW, NCH, CH)

    degp = _deg_kernel(dst3).reshape(NC, N)
    d0 = degp[0].reshape(N, 1)
    d1 = degp[1].reshape(N, 1)

    y1, dinv = pl.pallas_call(
        _tc1_body,
        out_shape=[
            jax.ShapeDtypeStruct((N, D_h), jnp.float32),
            jax.ShapeDtypeStruct((N, 1), jnp.float32),
        ],
    )(x, W1, d0, d1)

    r1 = _agg2(y1[:, :AD], y1[:, AD:], src3, dst3)
    agg1 = r1[:, :, :N, :].transpose(0, 2, 1, 3).reshape(NC, N, D_h)

    y2 = pl.pallas_call(
        _tc2_body,
        out_shape=jax.ShapeDtypeStruct((N, D_out), jnp.float32),
    )(agg1[0], agg1[1], y1, dinv, b1.reshape(1, D_h), W2)

    agg2 = _agg1(y2, src3, dst3)[:, 0, :N, :]

    out = pl.pallas_call(
        _tc3_body,
        out_shape=jax.ShapeDtypeStruct((N, D_out), jnp.float32),
    )(agg2[0], agg2[1], y2, dinv, b2.reshape(1, D_out))

    return out


# speedup vs baseline: 27.9919x; 1.7753x over previous
import functools

import jax
import jax.numpy as jnp
from jax import lax
from jax.experimental import pallas as pl
from jax.experimental.pallas import tpu as pltpu
from jax.experimental.pallas import tpu_sc as plsc

N = 10000
E = 320000
NC = 2
NS = 16
NW = NC * NS
EP = E // NW
CH = 128
NCH = 79
PADE = NCH * CH - EP
NPAD = 10240
ROWS_PER_TILE = NPAD // NS
AD = 64

_mesh = plsc.VectorSubcoreMesh(core_axis_name="c", subcore_axis_name="s")
_sc_params = pltpu.CompilerParams(use_tc_tiling_on_sc=False)


@functools.partial(
    pl.kernel,
    out_type=jax.ShapeDtypeStruct((NC * NPAD,), jnp.float32),
    mesh=_mesh,
    compiler_params=_sc_params,
    scratch_types=[
        pltpu.VMEM((NCH, CH), jnp.int32),
        pltpu.VMEM((CH,), jnp.float32),
        pltpu.VMEM((ROWS_PER_TILE,), jnp.float32),
        pltpu.VMEM_SHARED((NPAD,), jnp.float32),
    ],
)
def _deg_kernel(dst_hbm, out_hbm, didx, ones_v, zbuf, deg_sp):
    c = lax.axis_index("c")
    s = lax.axis_index("s")
    wid = c * NS + s

    def zstore(i, _):
        zbuf[pl.ds(i * 16, 16)] = jnp.zeros((16,), jnp.float32)
        return 0

    lax.fori_loop(0, ROWS_PER_TILE // 16, zstore, 0)

    def fill_ones(i, _):
        ones_v[pl.ds(i * 16, 16)] = jnp.ones((16,), jnp.float32)
        return 0

    lax.fori_loop(0, CH // 16, fill_ones, 0)

    pltpu.sync_copy(zbuf, deg_sp.at[pl.ds(s * ROWS_PER_TILE, ROWS_PER_TILE)])
    plsc.subcore_barrier()

    pltpu.sync_copy(dst_hbm.at[wid], didx)

    def body(j, _):
        pltpu.sync_copy(ones_v, deg_sp.at[didx.at[j]], add=True)
        return 0

    lax.fori_loop(0, NCH, body, 0)

    plsc.subcore_barrier()

    pltpu.sync_copy(deg_sp.at[pl.ds(s * ROWS_PER_TILE, ROWS_PER_TILE)], zbuf)
    pltpu.sync_copy(
        zbuf, out_hbm.at[pl.ds(c * NPAD + s * ROWS_PER_TILE, ROWS_PER_TILE)])


def _make_agg(P):
    zr = 128

    @functools.partial(
        pl.kernel,
        out_type=jax.ShapeDtypeStruct((NC, P, NPAD, AD), jnp.float32),
        mesh=_mesh,
        compiler_params=_sc_params,
        scratch_types=[
            pltpu.VMEM((NCH, CH), jnp.int32),
            pltpu.VMEM((NCH, CH), jnp.int32),
            pltpu.VMEM((CH, AD), jnp.float32),
            pltpu.VMEM((CH, AD), jnp.float32),
            pltpu.VMEM((zr, AD), jnp.float32),
            pltpu.VMEM_SHARED((NPAD, AD), jnp.float32),
            pltpu.SemaphoreType.DMA,
            pltpu.SemaphoreType.DMA,
        ],
    )
    def agg(*args):
        ys = args[:P]
        (src_hbm, dst_hbm, out_hbm,
         sidx, didx, rows0, rows1, zbuf, acc, sem0, sem1) = args[P:]
        c = lax.axis_index("c")
        s = lax.axis_index("s")
        wid = c * NS + s

        cols = AD // 16

        def zstore(t, _):
            zbuf[t // cols, pl.ds((t % cols) * 16, 16)] = jnp.zeros(
                (16,), jnp.float32)
            return 0

        lax.fori_loop(0, zr * cols, zstore, 0)

        pltpu.sync_copy(src_hbm.at[wid], sidx)
        pltpu.sync_copy(dst_hbm.at[wid], didx)

        for p in range(P):
            for k in range(ROWS_PER_TILE // zr):
                pltpu.sync_copy(
                    zbuf, acc.at[pl.ds(s * ROWS_PER_TILE + k * zr, zr)])
            plsc.subcore_barrier()

            y_hbm = ys[p]

            def gather_start(j, buf, sem):
                return pltpu.async_copy(y_hbm.at[sidx.at[j]], buf, sem)

            def gather_wait(j, buf, sem):
                pltpu.make_async_copy(y_hbm.at[sidx.at[j]], buf, sem).wait()

            def scat(j, buf):
                pltpu.sync_copy(buf, acc.at[didx.at[j]], add=True)

            gather_start(0, rows0, sem0)

            def body(k, _):
                j0 = 2 * k
                gather_start(j0 + 1, rows1, sem1)
                gather_wait(j0, rows0, sem0)
                scat(j0, rows0)
                gather_start(j0 + 2, rows0, sem0)
                gather_wait(j0 + 1, rows1, sem1)
                scat(j0 + 1, rows1)
                return 0

            lax.fori_loop(0, (NCH - 1) // 2, body, 0)

            gather_wait(NCH - 1, rows0, sem0)
            scat(NCH - 1, rows0)

            plsc.subcore_barrier()

            for k in range(ROWS_PER_TILE // zr):
                base = s * ROWS_PER_TILE + k * zr
                pltpu.sync_copy(acc.at[pl.ds(base, zr)], zbuf)
                pltpu.sync_copy(zbuf, out_hbm.at[c, p, pl.ds(base, zr)])
            if p + 1 < P:
                lax.fori_loop(0, zr * cols, zstore, 0)

    return agg


_agg2 = _make_agg(2)
_agg1 = _make_agg(1)


def _tc1_body(x_ref, w_ref, d0_ref, d1_ref, ya_ref, yb_ref, dinv_ref):
    deg = d0_ref[0:N, :] + d1_ref[0:N, :] + 1.0
    dinv = lax.rsqrt(deg)
    dinv_ref[...] = dinv
    xw = jnp.dot(x_ref[...], w_ref[...],
                 preferred_element_type=jnp.float32,
                 precision=lax.Precision.HIGHEST)
    y = xw * dinv
    ya_ref[...] = y[:, :AD]
    yb_ref[...] = y[:, AD:]


_SELU_SCALE = 1.0507009873554804934193349852946
_SELU_ALPHA = 1.6732632423543772848170429916717


def _tc2_body(q00_ref, q01_ref, q10_ref, q11_ref, ya_ref, yb_ref,
              dinv_ref, b1_ref, w2_ref, y2_ref):
    dinv = dinv_ref[...]
    agg = jnp.concatenate(
        [q00_ref[...] + q10_ref[...] + ya_ref[...],
         q01_ref[...] + q11_ref[...] + yb_ref[...]], axis=1)
    pre = dinv * agg + b1_ref[...]
    h = _SELU_SCALE * jnp.where(
        pre > 0.0, pre, _SELU_ALPHA * (jnp.exp(pre) - 1.0))
    hw = jnp.dot(h, w2_ref[...],
                 preferred_element_type=jnp.float32,
                 precision=lax.Precision.HIGHEST)
    y2_ref[...] = hw * dinv


def _tc3_body(q0_ref, q1_ref, y2_ref, dinv_ref, b2_ref, out_ref):
    o = (dinv_ref[...] * (q0_ref[...] + q1_ref[...] + y2_ref[...])
         + b2_ref[...])
    m = jnp.max(o, axis=1, keepdims=True)
    lse = m + jnp.log(jnp.sum(jnp.exp(o - m), axis=1, keepdims=True))
    out_ref[...] = o - lse


def kernel(x, edge_index, W1, b1, W2, b2):
    D_h = W1.shape[1]
    D_out = W2.shape[1]

    i = jnp.arange(PADE, dtype=jnp.int32)[None, :]
    w = jnp.arange(NW, dtype=jnp.int32)[:, None]
    pad_src = (i * 83 + w * 41) % N
    pad_dst = N + (i + w * 7) % (NPAD - N)
    src3 = jnp.concatenate(
        [edge_index[0].reshape(NW, EP), pad_src], axis=1).reshape(NW, NCH, CH)
    dst3 = jnp.concatenate(
        [edge_index[1].reshape(NW, EP), pad_dst], axis=1).reshape(NW, NCH, CH)

    degp = _deg_kernel(dst3)
    d0 = degp[:NPAD].reshape(NPAD, 1)
    d1 = degp[NPAD:].reshape(NPAD, 1)

    ya, yb, dinv = pl.pallas_call(
        _tc1_body,
        out_shape=[
            jax.ShapeDtypeStruct((N, AD), jnp.float32),
            jax.ShapeDtypeStruct((N, AD), jnp.float32),
            jax.ShapeDtypeStruct((N, 1), jnp.float32),
        ],
    )(x, W1, d0, d1)

    r1 = _agg2(ya, yb, src3, dst3)

    R = 2000
    _rows64 = pl.BlockSpec((R, AD), lambda i: (i, 0))
    _rows1 = pl.BlockSpec((R, 1), lambda i: (i, 0))

    y2 = pl.pallas_call(
        _tc2_body,
        grid=(N // R,),
        in_specs=[_rows64, _rows64, _rows64, _rows64, _rows64, _rows64,
                  _rows1,
                  pl.BlockSpec((1, D_h), lambda i: (0, 0)),
                  pl.BlockSpec((D_h, D_out), lambda i: (0, 0))],
        out_specs=pl.BlockSpec((R, D_out), lambda i: (i, 0)),
        out_shape=jax.ShapeDtypeStruct((N, D_out), jnp.float32),
    )(r1[0, 0], r1[0, 1], r1[1, 0], r1[1, 1], ya, yb,
      dinv, b1.reshape(1, D_h), W2)

    r2 = _agg1(y2, src3, dst3)

    out = pl.pallas_call(
        _tc3_body,
        grid=(N // R,),
        in_specs=[_rows64, _rows64, _rows64, _rows1,
                  pl.BlockSpec((1, D_out), lambda i: (0, 0))],
        out_specs=pl.BlockSpec((R, D_out), lambda i: (i, 0)),
        out_shape=jax.ShapeDtypeStruct((N, D_out), jnp.float32),
    )(r2[0, 0], r2[1, 0], y2, dinv, b2.reshape(1, D_out))

    return out
